# baseline (speedup 1.0000x reference)
import functools

import jax
import jax.numpy as jnp
from jax import lax
from jax.experimental import pallas as pl
from jax.experimental.pallas import tpu as pltpu
from jax.experimental.pallas import tpu_sc as plsc

BATCH = 16384
EMB = 64
NUM_WORKERS = 32
ROWS_PER_WORKER = BATCH // NUM_WORKERS
IDX_CHUNK = 128
NUM_CHUNKS = ROWS_PER_WORKER // IDX_CHUNK

_sc_mesh = plsc.VectorSubcoreMesh(core_axis_name="c", subcore_axis_name="s")


@functools.partial(
    pl.kernel,
    mesh=_sc_mesh,
    compiler_params=pltpu.CompilerParams(use_tc_tiling_on_sc=False),
    out_type=jax.ShapeDtypeStruct((BATCH, EMB), jnp.float32),
    scratch_types=[
        pltpu.VMEM((NUM_CHUNKS, IDX_CHUNK), jnp.int32),
        pltpu.VMEM((ROWS_PER_WORKER, EMB), jnp.float32),
        pltpu.SemaphoreType.DMA,
    ],
)
def _sc_gather(ids_hbm, tab_hbm, out_hbm, idx_v, rows_v, sem):
    wid = lax.axis_index("s") * 2 + lax.axis_index("c")
    base = wid * ROWS_PER_WORKER
    pltpu.sync_copy(ids_hbm.at[wid], idx_v)
    copies = []
    for j in range(NUM_CHUNKS):
        dst = rows_v.at[pl.ds(j * IDX_CHUNK, IDX_CHUNK)]
        copies.append(pltpu.async_copy(tab_hbm.at[idx_v.at[j]], dst, sem))
    for c in copies:
        c.wait()
    pltpu.sync_copy(rows_v, out_hbm.at[pl.ds(base, ROWS_PER_WORKER)])


_TCHUNK = 8192


def _transpose_body(x_ref, o_ref):
    o_ref[...] = x_ref[...].T


def _transpose_tc(t):
    n = t.shape[1]
    return pl.pallas_call(
        _transpose_body,
        grid=(pl.cdiv(n, _TCHUNK),),
        in_specs=[pl.BlockSpec((EMB, _TCHUNK), lambda j: (0, j))],
        out_specs=pl.BlockSpec((_TCHUNK, EMB), lambda j: (j, 0)),
        out_shape=jax.ShapeDtypeStruct((n, EMB), jnp.float32),
    )(t)


def _mlp_body(u_ref, i_ref, w1_ref, b1_ref, w2_ref, b2_ref, o_ref):
    prod = u_ref[...] * i_ref[...]
    h = jnp.dot(prod, w1_ref[...], preferred_element_type=jnp.float32)
    h = jnp.maximum(h + b1_ref[...], 0.0)
    o = jnp.sum(h * w2_ref[...], axis=1) + b2_ref[0, 0]
    o_ref[...] = jax.nn.sigmoid(o)


def kernel(user_ids, item_ids, user_table, item_table, W1, b1, W2, b2):
    uid = user_ids.astype(jnp.int32).reshape(NUM_WORKERS, NUM_CHUNKS, IDX_CHUNK)
    iid = item_ids.astype(jnp.int32).reshape(NUM_WORKERS, NUM_CHUNKS, IDX_CHUNK)

    ut = _transpose_tc(user_table.T)
    u_emb = _sc_gather(uid, ut)
    it = _transpose_tc(item_table.T)
    i_emb = _sc_gather(iid, it)

    blk = 2048
    out = pl.pallas_call(
        _mlp_body,
        grid=(BATCH // blk,),
        in_specs=[
            pl.BlockSpec((blk, EMB), lambda b: (b, 0)),
            pl.BlockSpec((blk, EMB), lambda b: (b, 0)),
            pl.BlockSpec((EMB, 32), lambda b: (0, 0)),
            pl.BlockSpec((1, 32), lambda b: (0, 0)),
            pl.BlockSpec((1, 32), lambda b: (0, 0)),
            pl.BlockSpec((1, 1), lambda b: (0, 0)),
        ],
        out_specs=pl.BlockSpec((blk,), lambda b: (b,)),
        out_shape=jax.ShapeDtypeStruct((BATCH,), jnp.float32),
    )(u_emb, i_emb, W1, b1.reshape(1, 32), W2.reshape(1, 32),
      b2.reshape(1, 1))
    return out


# speedup vs baseline: 2.3718x; 2.3718x over previous
import functools

import jax
import jax.numpy as jnp
from jax import lax
from jax.experimental import pallas as pl
from jax.experimental.pallas import tpu as pltpu
from jax.experimental.pallas import tpu_sc as plsc

BATCH = 16384
EMB = 64
NUM_WORKERS = 32
ROWS_PER_WORKER = BATCH // NUM_WORKERS
IDX_CHUNK = 128
NUM_CHUNKS = ROWS_PER_WORKER // IDX_CHUNK

_LANES = 8192
_HALF = _LANES // 2
_NBLK = 123
_TROWS = _NBLK * _HALF

_sc_mesh = plsc.VectorSubcoreMesh(core_axis_name="c", subcore_axis_name="s")


@functools.partial(
    pl.kernel,
    mesh=_sc_mesh,
    compiler_params=pltpu.CompilerParams(use_tc_tiling_on_sc=False),
    out_type=jax.ShapeDtypeStruct((BATCH, 2 * EMB), jnp.float32),
    scratch_types=[
        pltpu.VMEM((NUM_CHUNKS, IDX_CHUNK), jnp.int32),
        pltpu.VMEM((ROWS_PER_WORKER, 2 * EMB), jnp.float32),
        pltpu.SemaphoreType.DMA,
    ],
)
def _sc_gather(ids_hbm, tab_hbm, out_hbm, idx_v, rows_v, sem):
    wid = lax.axis_index("s") * 2 + lax.axis_index("c")
    base = wid * ROWS_PER_WORKER
    pltpu.sync_copy(ids_hbm.at[wid], idx_v)
    copies = []
    for j in range(NUM_CHUNKS):
        dst = rows_v.at[pl.ds(j * IDX_CHUNK, IDX_CHUNK)]
        copies.append(pltpu.async_copy(tab_hbm.at[idx_v.at[j]], dst, sem))
    for c in copies:
        c.wait()
    pltpu.sync_copy(rows_v, out_hbm.at[pl.ds(base, ROWS_PER_WORKER)])


def _transpose_body(x_ref, o_ref):
    o_ref[:, 0:EMB] = x_ref[:, 0:_HALF].T
    o_ref[:, EMB:2 * EMB] = x_ref[:, _HALF:_LANES].T


def _transpose_tc(t):
    return pl.pallas_call(
        _transpose_body,
        grid=(_NBLK,),
        in_specs=[pl.BlockSpec((EMB, _LANES), lambda j: (0, j))],
        out_specs=pl.BlockSpec((_HALF, 2 * EMB), lambda j: (j, 0)),
        out_shape=jax.ShapeDtypeStruct((_TROWS, 2 * EMB), jnp.float32),
    )(t)


def _mlp_body(u_ref, i_ref, uh_ref, ih_ref, w1_ref, b1_ref, w2_ref, b2_ref,
              o_ref):
    uh = uh_ref[...] > 0
    ih = ih_ref[...] > 0
    u = jnp.where(uh, u_ref[:, EMB:2 * EMB], u_ref[:, 0:EMB])
    i = jnp.where(ih, i_ref[:, EMB:2 * EMB], i_ref[:, 0:EMB])
    prod = u * i
    h = jnp.dot(prod, w1_ref[...], preferred_element_type=jnp.float32)
    h = jnp.maximum(h + b1_ref[...], 0.0)
    o = jnp.sum(h * w2_ref[...], axis=1) + b2_ref[0, 0]
    o_ref[...] = jax.nn.sigmoid(o)


def kernel(user_ids, item_ids, user_table, item_table, W1, b1, W2, b2):
    uid = user_ids.astype(jnp.int32)
    iid = item_ids.astype(jnp.int32)
    ug = ((uid >> 13) << 12) | (uid & 4095)
    ig = ((iid >> 13) << 12) | (iid & 4095)
    uh = ((uid >> 12) & 1).reshape(BATCH, 1)
    ih = ((iid >> 12) & 1).reshape(BATCH, 1)
    ug = ug.reshape(NUM_WORKERS, NUM_CHUNKS, IDX_CHUNK)
    ig = ig.reshape(NUM_WORKERS, NUM_CHUNKS, IDX_CHUNK)

    ut = _transpose_tc(user_table.T)
    u_emb = _sc_gather(ug, ut)
    it = _transpose_tc(item_table.T)
    i_emb = _sc_gather(ig, it)

    blk = 2048
    out = pl.pallas_call(
        _mlp_body,
        grid=(BATCH // blk,),
        in_specs=[
            pl.BlockSpec((blk, 2 * EMB), lambda b: (b, 0)),
            pl.BlockSpec((blk, 2 * EMB), lambda b: (b, 0)),
            pl.BlockSpec((blk, 1), lambda b: (b, 0)),
            pl.BlockSpec((blk, 1), lambda b: (b, 0)),
            pl.BlockSpec((EMB, 32), lambda b: (0, 0)),
            pl.BlockSpec((1, 32), lambda b: (0, 0)),
            pl.BlockSpec((1, 32), lambda b: (0, 0)),
            pl.BlockSpec((1, 1), lambda b: (0, 0)),
        ],
        out_specs=pl.BlockSpec((blk,), lambda b: (b,)),
        out_shape=jax.ShapeDtypeStruct((BATCH,), jnp.float32),
    )(u_emb, i_emb, uh, ih, W1, b1.reshape(1, 32), W2.reshape(1, 32),
      b2.reshape(1, 1))
    return out
